# baseline (speedup 1.0000x reference)
import functools

import jax
import jax.numpy as jnp
from jax import lax
from jax.experimental import pallas as pl
from jax.experimental.pallas import tpu as pltpu
from jax.experimental.pallas import tpu_sc as plsc

B = 8
P = 1024
K = 20
KP = 24
BIG = 3.0e38
BN_SC = float(1.0 / (1.0 + 1e-5) ** 0.5)

H = 2
CC = B // H
NH = CC * P

NC = 2
NS = 16
NW = NC * NS
RPW = NH // NW
CHS = 32
NCH = RPW // CHS

BLK = 256
NBLK = NH // BLK


def _dot(a, b):
    return lax.dot_general(a, b, (((1,), (0,)), ((), ())),
                           preferred_element_type=jnp.float32)


def _dot_t(a, b):
    return lax.dot_general(a, b, (((1,), (1,)), ((), ())),
                           preferred_element_type=jnp.float32)


def _layer_a_body(*refs):
    f_ref, wt, bias, pt_out, idx_out = refs
    f = f_ref[...]
    cloud = pl.program_id(0)

    pt_out[...] = _dot(f, wt[...]) + bias[...]

    g = _dot_t(f, f)
    sq = jnp.sum(f * f, axis=1)
    d2 = sq[:, None] + sq[None, :] - 2.0 * g
    col = lax.broadcasted_iota(jnp.int32, (P, P), 1)
    off = cloud * P
    x = d2
    for k in range(K):
        amin = jnp.argmin(x, axis=1).astype(jnp.int32)
        idx_out[pl.ds(k, 1), :] = (amin + off)[None, :]
        x = jnp.where(col == amin[:, None], BIG, x)
    idx_out[pl.ds(K, KP - K), :] = jnp.zeros((KP - K, P), jnp.int32)


def _tc_layer_a(f, wt, bias, dp, op):
    in_specs = [
        pl.BlockSpec((P, dp), lambda i: (i, 0)),
        pl.BlockSpec((dp, op), lambda i: (0, 0)),
        pl.BlockSpec((1, op), lambda i: (0, 0)),
    ]
    out_specs = [
        pl.BlockSpec((P, op), lambda i: (i, 0)),
        pl.BlockSpec((KP, P), lambda i: (0, i)),
    ]
    out_shape = [
        jax.ShapeDtypeStruct((NH, op), jnp.float32),
        jax.ShapeDtypeStruct((KP, NH), jnp.int32),
    ]
    return pl.pallas_call(
        _layer_a_body, grid=(CC,), in_specs=in_specs, out_specs=out_specs,
        out_shape=out_shape,
    )(f, wt, bias)


def _sc_gather(f, idx_t, dp):
    mesh = plsc.VectorSubcoreMesh(core_axis_name="c", subcore_axis_name="s")

    @functools.partial(
        pl.kernel, mesh=mesh,
        out_type=jax.ShapeDtypeStruct((K, NH, dp), jnp.float32),
        scratch_types=[
            pltpu.VMEM((KP, RPW), jnp.int32),
            pltpu.VMEM((K, CHS, dp), jnp.float32),
            pltpu.SemaphoreType.DMA,
            pltpu.SemaphoreType.DMA,
        ],
    )
    def sck(f_hbm, idx_hbm, e_hbm, idx_v, gbuf, gsem, wsem):
        wid = lax.axis_index("s") * NC + lax.axis_index("c")
        base = wid * RPW
        pltpu.sync_copy(idx_hbm.at[:, pl.ds(base, RPW)], idx_v)

        def chunk(c, carry):
            r0 = c * CHS
            gets = [
                pltpu.async_copy(
                    f_hbm.at[idx_v.at[k, pl.ds(r0, CHS)]], gbuf.at[k], gsem)
                for k in range(K)
            ]
            puts = []
            for k in range(K):
                gets[k].wait()
                puts.append(pltpu.async_copy(
                    gbuf.at[k], e_hbm.at[k, pl.ds(base + r0, CHS)], wsem))
            for p in puts:
                p.wait()
            return carry

        lax.fori_loop(0, NCH, chunk, 0)

    return sck(f, idx_t)


def _layer_b_body(*refs):
    e_ref, f_ref, pt_ref, wb, out_ref = refs
    f = f_ref[...]
    w = wb[...]
    acc = None
    for k in range(K):
        ed = _dot(e_ref[k] - f, w)
        acc = ed if acc is None else jnp.maximum(acc, ed)
    out_ref[...] = acc + pt_ref[...]


def _tc_layer_b(e, f, pt, wb, dp, op):
    in_specs = [
        pl.BlockSpec((K, BLK, dp), lambda j: (0, j, 0)),
        pl.BlockSpec((BLK, dp), lambda j: (j, 0)),
        pl.BlockSpec((BLK, op), lambda j: (j, 0)),
        pl.BlockSpec((dp, op), lambda j: (0, 0)),
    ]
    return pl.pallas_call(
        _layer_b_body, grid=(NBLK,), in_specs=in_specs,
        out_specs=pl.BlockSpec((BLK, op), lambda j: (j, 0)),
        out_shape=jax.ShapeDtypeStruct((NH, op), jnp.float32),
    )(e, f, pt, wb)


def _fc1pool_body(*refs):
    f1, f2, f3, f4, w1, w2, w3, w4, bf1, out = refs
    cloud = pl.program_id(0)
    h = (_dot(f1[...], w1[...]) + _dot(f2[...], w2[...])
         + _dot(f3[...], w3[...]) + _dot(f4[...], w4[...]) + bf1[...])
    out[pl.ds(cloud, 1), :] = jnp.max(h, axis=0)[None, :]


def _fc1pool(fs, wf1_splits, bf1):
    dims = (128, 128, 128, 256)
    in_specs = [pl.BlockSpec((P, d), lambda i: (i, 0)) for d in dims]
    for d in dims:
        in_specs.append(pl.BlockSpec((d, 1024), lambda i: (0, 0)))
    in_specs.append(pl.BlockSpec((1, 1024), lambda i: (0, 0)))
    return pl.pallas_call(
        _fc1pool_body, grid=(CC,), in_specs=in_specs,
        out_specs=pl.BlockSpec((CC, 1024), lambda i: (0, 0)),
        out_shape=jax.ShapeDtypeStruct((CC, 1024), jnp.float32),
    )(*fs, *wf1_splits, bf1)


def _cls_body(*refs):
    pool_ref, wa, ba, wb, bb, wc, bc, out = refs
    pool = pool_ref[...]
    h2 = jax.nn.relu((_dot(pool, wa[...]) + ba[...]) * BN_SC)
    h3 = jax.nn.relu((_dot(h2, wb[...]) + bb[...]) * BN_SC)
    logits = _dot(h3, wc[...]) + bc[...]
    mx = jnp.max(logits, axis=1, keepdims=True)
    lse = mx + jnp.log(jnp.sum(jnp.exp(logits - mx), axis=1, keepdims=True))
    out[...] = logits - lse


def _classifier(pooled, wa, ba, wb, bb, wc, bc, nclass):
    return pl.pallas_call(
        _cls_body,
        out_shape=jax.ShapeDtypeStruct((B, nclass), jnp.float32),
    )(pooled, wa, ba, wb, bb, wc, bc)


def kernel(pos, x, W1, b1, W2, b2, W3, b3, W4, b4, Wf1, bf1,
           Wa, ba, Wb, bb, Wc, bc, batch):
    x0 = jnp.concatenate([pos, x], axis=1).astype(jnp.float32)
    x0 = jnp.pad(x0, ((0, 0), (0, 122)))

    def prep_w(W, b, dt, op):
        ot = W.shape[1]
        pad = ((0, 128 - dt), (0, op - ot))
        return (jnp.pad(W[:dt], pad), jnp.pad(W[dt:], pad),
                jnp.pad(b, (0, op - ot))[None, :])

    dims = ((6, 128), (64, 128), (64, 128), (128, 256))
    weights = ((W1, b1), (W2, b2), (W3, b3), (W4, b4))
    prepped = [prep_w(W, b, dt, op) for (dt, op), (W, b) in zip(dims, weights)]

    rpad = ((0, 64), (0, 0))
    wf1_splits = (jnp.pad(Wf1[0:64], rpad), jnp.pad(Wf1[64:128], rpad),
                  Wf1[128:256], Wf1[256:512])
    nclass = Wc.shape[1]

    pooled_parts = []
    for h in range(H):
        f = x0[h * NH:(h + 1) * NH]
        fs = []
        for (dt, op), (wt, wbot, bias) in zip(dims, prepped):
            pt, idx_t = _tc_layer_a(f, wt, bias, 128, op)
            e = _sc_gather(f, idx_t, 128)
            f = _tc_layer_b(e, f, pt, wbot, 128, op)
            fs.append(f)
        pooled_parts.append(_fc1pool(fs, wf1_splits, bf1[None, :]))

    pooled = jnp.concatenate(pooled_parts, axis=0)
    return _classifier(pooled, Wa, ba[None, :], Wb, bb[None, :],
                       Wc, bc[None, :], nclass)


# speedup vs baseline: 1.0094x; 1.0094x over previous
import functools

import jax
import jax.numpy as jnp
from jax import lax
from jax.experimental import pallas as pl
from jax.experimental.pallas import tpu as pltpu
from jax.experimental.pallas import tpu_sc as plsc

B = 8
P = 1024
K = 20
KP = 24
BIG = 3.0e38
BN_SC = float(1.0 / (1.0 + 1e-5) ** 0.5)

H = 1
CC = B // H
NH = CC * P

NC = 2
NS = 16
NW = NC * NS
RPW = NH // NW
CHS = 32
NCH = RPW // CHS

BLK = 256
NBLK = NH // BLK


def _dot(a, b):
    return lax.dot_general(a, b, (((1,), (0,)), ((), ())),
                           preferred_element_type=jnp.float32)


def _dot_t(a, b):
    return lax.dot_general(a, b, (((1,), (1,)), ((), ())),
                           preferred_element_type=jnp.float32)


def _layer_a_body(*refs):
    f_ref, wt, bias, pt_out, idx_out = refs
    f = f_ref[...]
    cloud = pl.program_id(0)

    pt_out[...] = _dot(f, wt[...]) + bias[...]

    g = _dot_t(f, f)
    sq = jnp.sum(f * f, axis=1)
    d2 = sq[:, None] + sq[None, :] - 2.0 * g
    col = lax.broadcasted_iota(jnp.int32, (P, P), 1)
    off = cloud * P
    x = d2
    amin = jnp.argmin(x, axis=1).astype(jnp.int32)
    idx_out[pl.ds(0, 1), :] = (amin + off)[None, :]
    for k in range(1, K):
        x = jnp.where(col == amin[:, None], BIG, x)
        amin = jnp.argmin(x, axis=1).astype(jnp.int32)
        idx_out[pl.ds(k, 1), :] = (amin + off)[None, :]
    idx_out[pl.ds(K, KP - K), :] = jnp.zeros((KP - K, P), jnp.int32)


def _tc_layer_a(f, wt, bias, dp, op):
    in_specs = [
        pl.BlockSpec((P, dp), lambda i: (i, 0)),
        pl.BlockSpec((dp, op), lambda i: (0, 0)),
        pl.BlockSpec((1, op), lambda i: (0, 0)),
    ]
    out_specs = [
        pl.BlockSpec((P, op), lambda i: (i, 0)),
        pl.BlockSpec((KP, P), lambda i: (0, i)),
    ]
    out_shape = [
        jax.ShapeDtypeStruct((NH, op), jnp.float32),
        jax.ShapeDtypeStruct((KP, NH), jnp.int32),
    ]
    return pl.pallas_call(
        _layer_a_body, grid=(CC,), in_specs=in_specs, out_specs=out_specs,
        out_shape=out_shape,
    )(f, wt, bias)


def _sc_gather(f, idx_t, dp):
    mesh = plsc.VectorSubcoreMesh(core_axis_name="c", subcore_axis_name="s")

    @functools.partial(
        pl.kernel, mesh=mesh,
        out_type=jax.ShapeDtypeStruct((K, NH, dp), jnp.float32),
        scratch_types=[
            pltpu.VMEM((KP, RPW), jnp.int32),
            pltpu.VMEM((K, CHS, dp), jnp.float32),
            pltpu.SemaphoreType.DMA,
            pltpu.SemaphoreType.DMA,
        ],
    )
    def sck(f_hbm, idx_hbm, e_hbm, idx_v, gbuf, gsem, wsem):
        wid = lax.axis_index("s") * NC + lax.axis_index("c")
        base = wid * RPW
        pltpu.sync_copy(idx_hbm.at[:, pl.ds(base, RPW)], idx_v)

        def chunk(c, carry):
            r0 = c * CHS
            gets = [
                pltpu.async_copy(
                    f_hbm.at[idx_v.at[k, pl.ds(r0, CHS)]], gbuf.at[k], gsem)
                for k in range(K)
            ]
            puts = []
            for k in range(K):
                gets[k].wait()
                puts.append(pltpu.async_copy(
                    gbuf.at[k], e_hbm.at[k, pl.ds(base + r0, CHS)], wsem))
            for p in puts:
                p.wait()
            return carry

        lax.fori_loop(0, NCH, chunk, 0)

    return sck(f, idx_t)


def _layer_b_body(*refs):
    e_ref, f_ref, pt_ref, wb, out_ref = refs
    f = f_ref[...]
    w = wb[...]
    acc = None
    for k in range(K):
        ed = _dot(e_ref[k] - f, w)
        acc = ed if acc is None else jnp.maximum(acc, ed)
    out_ref[...] = acc + pt_ref[...]


def _tc_layer_b(e, f, pt, wb, dp, op):
    in_specs = [
        pl.BlockSpec((K, BLK, dp), lambda j: (0, j, 0)),
        pl.BlockSpec((BLK, dp), lambda j: (j, 0)),
        pl.BlockSpec((BLK, op), lambda j: (j, 0)),
        pl.BlockSpec((dp, op), lambda j: (0, 0)),
    ]
    return pl.pallas_call(
        _layer_b_body, grid=(NBLK,), in_specs=in_specs,
        out_specs=pl.BlockSpec((BLK, op), lambda j: (j, 0)),
        out_shape=jax.ShapeDtypeStruct((NH, op), jnp.float32),
    )(e, f, pt, wb)


def _fc1pool_body(*refs):
    f1, f2, f3, f4, w1, w2, w3, w4, bf1, out = refs
    cloud = pl.program_id(0)
    h = (_dot(f1[...], w1[...]) + _dot(f2[...], w2[...])
         + _dot(f3[...], w3[...]) + _dot(f4[...], w4[...]) + bf1[...])
    out[pl.ds(cloud, 1), :] = jnp.max(h, axis=0)[None, :]


def _fc1pool(fs, wf1_splits, bf1):
    dims = (128, 128, 128, 256)
    in_specs = [pl.BlockSpec((P, d), lambda i: (i, 0)) for d in dims]
    for d in dims:
        in_specs.append(pl.BlockSpec((d, 1024), lambda i: (0, 0)))
    in_specs.append(pl.BlockSpec((1, 1024), lambda i: (0, 0)))
    return pl.pallas_call(
        _fc1pool_body, grid=(CC,), in_specs=in_specs,
        out_specs=pl.BlockSpec((CC, 1024), lambda i: (0, 0)),
        out_shape=jax.ShapeDtypeStruct((CC, 1024), jnp.float32),
    )(*fs, *wf1_splits, bf1)


def _cls_body(*refs):
    pool_ref, wa, ba, wb, bb, wc, bc, out = refs
    pool = pool_ref[...]
    h2 = jax.nn.relu((_dot(pool, wa[...]) + ba[...]) * BN_SC)
    h3 = jax.nn.relu((_dot(h2, wb[...]) + bb[...]) * BN_SC)
    logits = _dot(h3, wc[...]) + bc[...]
    mx = jnp.max(logits, axis=1, keepdims=True)
    lse = mx + jnp.log(jnp.sum(jnp.exp(logits - mx), axis=1, keepdims=True))
    out[...] = logits - lse


def _classifier(pooled, wa, ba, wb, bb, wc, bc, nclass):
    return pl.pallas_call(
        _cls_body,
        out_shape=jax.ShapeDtypeStruct((B, nclass), jnp.float32),
    )(pooled, wa, ba, wb, bb, wc, bc)


def kernel(pos, x, W1, b1, W2, b2, W3, b3, W4, b4, Wf1, bf1,
           Wa, ba, Wb, bb, Wc, bc, batch):
    x0 = jnp.concatenate([pos, x], axis=1).astype(jnp.float32)
    x0 = jnp.pad(x0, ((0, 0), (0, 122)))

    def prep_w(W, b, dt, op):
        ot = W.shape[1]
        pad = ((0, 128 - dt), (0, op - ot))
        return (jnp.pad(W[:dt], pad), jnp.pad(W[dt:], pad),
                jnp.pad(b, (0, op - ot))[None, :])

    dims = ((6, 128), (64, 128), (64, 128), (128, 256))
    weights = ((W1, b1), (W2, b2), (W3, b3), (W4, b4))
    prepped = [prep_w(W, b, dt, op) for (dt, op), (W, b) in zip(dims, weights)]

    rpad = ((0, 64), (0, 0))
    wf1_splits = (jnp.pad(Wf1[0:64], rpad), jnp.pad(Wf1[64:128], rpad),
                  Wf1[128:256], Wf1[256:512])
    nclass = Wc.shape[1]

    pooled_parts = []
    for h in range(H):
        f = x0[h * NH:(h + 1) * NH]
        fs = []
        for (dt, op), (wt, wbot, bias) in zip(dims, prepped):
            pt, idx_t = _tc_layer_a(f, wt, bias, 128, op)
            e = _sc_gather(f, idx_t, 128)
            f = _tc_layer_b(e, f, pt, wbot, 128, op)
            fs.append(f)
        pooled_parts.append(_fc1pool(fs, wf1_splits, bf1[None, :]))

    pooled = jnp.concatenate(pooled_parts, axis=0)
    return _classifier(pooled, Wa, ba[None, :], Wb, bb[None, :],
                       Wc, bc[None, :], nclass)


# speedup vs baseline: 1.0128x; 1.0034x over previous
import functools

import jax
import jax.numpy as jnp
from jax import lax
from jax.experimental import pallas as pl
from jax.experimental.pallas import tpu as pltpu
from jax.experimental.pallas import tpu_sc as plsc

B = 8
P = 1024
K = 20
KP = 24
BIG = 3.0e38
BN_SC = float(1.0 / (1.0 + 1e-5) ** 0.5)

H = 1
CC = B // H
NH = CC * P

NC = 2
NS = 16
NW = NC * NS
RPW = NH // NW
CHS = 16
NCH = RPW // CHS

BLK = 256
NBLK = NH // BLK


def _dot(a, b):
    return lax.dot_general(a, b, (((1,), (0,)), ((), ())),
                           preferred_element_type=jnp.float32)


def _dot_t(a, b):
    return lax.dot_general(a, b, (((1,), (1,)), ((), ())),
                           preferred_element_type=jnp.float32)


def _layer_a_body(*refs):
    f_ref, wt, bias, pt_out, idx_out = refs
    f = f_ref[...]
    cloud = pl.program_id(0)

    pt_out[...] = _dot(f, wt[...]) + bias[...]

    g = _dot_t(f, f)
    sq = jnp.sum(f * f, axis=1)
    d2 = sq[:, None] + sq[None, :] - 2.0 * g
    col = lax.broadcasted_iota(jnp.int32, (P, P), 1)
    off = cloud * P
    x = d2
    amin = jnp.argmin(x, axis=1).astype(jnp.int32)
    idx_out[pl.ds(0, 1), :] = (amin + off)[None, :]
    for k in range(1, K):
        x = jnp.where(col == amin[:, None], BIG, x)
        amin = jnp.argmin(x, axis=1).astype(jnp.int32)
        idx_out[pl.ds(k, 1), :] = (amin + off)[None, :]
    idx_out[pl.ds(K, KP - K), :] = jnp.zeros((KP - K, P), jnp.int32)


def _tc_layer_a(f, wt, bias, dp, op):
    in_specs = [
        pl.BlockSpec((P, dp), lambda i: (i, 0)),
        pl.BlockSpec((dp, op), lambda i: (0, 0)),
        pl.BlockSpec((1, op), lambda i: (0, 0)),
    ]
    out_specs = [
        pl.BlockSpec((P, op), lambda i: (i, 0)),
        pl.BlockSpec((KP, P), lambda i: (0, i)),
    ]
    out_shape = [
        jax.ShapeDtypeStruct((NH, op), jnp.float32),
        jax.ShapeDtypeStruct((KP, NH), jnp.int32),
    ]
    return pl.pallas_call(
        _layer_a_body, grid=(CC,), in_specs=in_specs, out_specs=out_specs,
        out_shape=out_shape,
    )(f, wt, bias)


def _sc_gather(f, idx_t, dp):
    mesh = plsc.VectorSubcoreMesh(core_axis_name="c", subcore_axis_name="s")

    @functools.partial(
        pl.kernel, mesh=mesh,
        out_type=jax.ShapeDtypeStruct((K, NH, dp), jnp.float32),
        scratch_types=[
            pltpu.VMEM((KP, RPW), jnp.int32),
            pltpu.VMEM((2, K, CHS, dp), jnp.float32),
            pltpu.SemaphoreType.DMA,
            pltpu.SemaphoreType.DMA,
        ],
    )
    def sck(f_hbm, idx_hbm, e_hbm, idx_v, gbuf, gsem, wsem):
        wid = lax.axis_index("s") * NC + lax.axis_index("c")
        base = wid * RPW
        pltpu.sync_copy(idx_hbm.at[:, pl.ds(base, RPW)], idx_v)

        def issue_gets(c, b):
            for k in range(K):
                pltpu.async_copy(
                    f_hbm.at[idx_v.at[k, pl.ds(c * CHS, CHS)]],
                    gbuf.at[b, k], gsem)

        def wait_gets(b):
            for k in range(K):
                pltpu.make_async_copy(
                    f_hbm.at[pl.ds(0, CHS)], gbuf.at[b, k], gsem).wait()

        def issue_puts(c, b):
            for k in range(K):
                pltpu.async_copy(
                    gbuf.at[b, k], e_hbm.at[k, pl.ds(base + c * CHS, CHS)],
                    wsem)

        def wait_puts(b):
            for k in range(K):
                pltpu.make_async_copy(
                    gbuf.at[b, k], e_hbm.at[k, pl.ds(0, CHS)], wsem).wait()

        issue_gets(0, 0)

        def body(i, carry):
            c0 = i * 2

            @pl.when(c0 > 0)
            def _():
                wait_puts(1)

            issue_gets(c0 + 1, 1)
            wait_gets(0)
            issue_puts(c0, 0)
            wait_puts(0)

            @pl.when(c0 + 2 < NCH)
            def _():
                issue_gets(c0 + 2, 0)

            wait_gets(1)
            issue_puts(c0 + 1, 1)
            return carry

        lax.fori_loop(0, NCH // 2, body, 0)
        wait_puts(1)

    return sck(f, idx_t)


def _layer_b_body(*refs):
    e_ref, f_ref, pt_ref, wb, out_ref = refs
    f = f_ref[...]
    w = wb[...]
    acc = None
    for k in range(K):
        ed = _dot(e_ref[k] - f, w)
        acc = ed if acc is None else jnp.maximum(acc, ed)
    out_ref[...] = acc + pt_ref[...]


def _tc_layer_b(e, f, pt, wb, dp, op):
    in_specs = [
        pl.BlockSpec((K, BLK, dp), lambda j: (0, j, 0)),
        pl.BlockSpec((BLK, dp), lambda j: (j, 0)),
        pl.BlockSpec((BLK, op), lambda j: (j, 0)),
        pl.BlockSpec((dp, op), lambda j: (0, 0)),
    ]
    return pl.pallas_call(
        _layer_b_body, grid=(NBLK,), in_specs=in_specs,
        out_specs=pl.BlockSpec((BLK, op), lambda j: (j, 0)),
        out_shape=jax.ShapeDtypeStruct((NH, op), jnp.float32),
    )(e, f, pt, wb)


def _fc1pool_body(*refs):
    f1, f2, f3, f4, w1, w2, w3, w4, bf1, out = refs
    cloud = pl.program_id(0)
    h = (_dot(f1[...], w1[...]) + _dot(f2[...], w2[...])
         + _dot(f3[...], w3[...]) + _dot(f4[...], w4[...]) + bf1[...])
    out[pl.ds(cloud, 1), :] = jnp.max(h, axis=0)[None, :]


def _fc1pool(fs, wf1_splits, bf1):
    dims = (128, 128, 128, 256)
    in_specs = [pl.BlockSpec((P, d), lambda i: (i, 0)) for d in dims]
    for d in dims:
        in_specs.append(pl.BlockSpec((d, 1024), lambda i: (0, 0)))
    in_specs.append(pl.BlockSpec((1, 1024), lambda i: (0, 0)))
    return pl.pallas_call(
        _fc1pool_body, grid=(CC,), in_specs=in_specs,
        out_specs=pl.BlockSpec((CC, 1024), lambda i: (0, 0)),
        out_shape=jax.ShapeDtypeStruct((CC, 1024), jnp.float32),
    )(*fs, *wf1_splits, bf1)


def _cls_body(*refs):
    pool_ref, wa, ba, wb, bb, wc, bc, out = refs
    pool = pool_ref[...]
    h2 = jax.nn.relu((_dot(pool, wa[...]) + ba[...]) * BN_SC)
    h3 = jax.nn.relu((_dot(h2, wb[...]) + bb[...]) * BN_SC)
    logits = _dot(h3, wc[...]) + bc[...]
    mx = jnp.max(logits, axis=1, keepdims=True)
    lse = mx + jnp.log(jnp.sum(jnp.exp(logits - mx), axis=1, keepdims=True))
    out[...] = logits - lse


def _classifier(pooled, wa, ba, wb, bb, wc, bc, nclass):
    return pl.pallas_call(
        _cls_body,
        out_shape=jax.ShapeDtypeStruct((B, nclass), jnp.float32),
    )(pooled, wa, ba, wb, bb, wc, bc)


def kernel(pos, x, W1, b1, W2, b2, W3, b3, W4, b4, Wf1, bf1,
           Wa, ba, Wb, bb, Wc, bc, batch):
    x0 = jnp.concatenate([pos, x], axis=1).astype(jnp.float32)
    x0 = jnp.pad(x0, ((0, 0), (0, 122)))

    def prep_w(W, b, dt, op):
        ot = W.shape[1]
        pad = ((0, 128 - dt), (0, op - ot))
        return (jnp.pad(W[:dt], pad), jnp.pad(W[dt:], pad),
                jnp.pad(b, (0, op - ot))[None, :])

    dims = ((6, 128), (64, 128), (64, 128), (128, 256))
    weights = ((W1, b1), (W2, b2), (W3, b3), (W4, b4))
    prepped = [prep_w(W, b, dt, op) for (dt, op), (W, b) in zip(dims, weights)]

    rpad = ((0, 64), (0, 0))
    wf1_splits = (jnp.pad(Wf1[0:64], rpad), jnp.pad(Wf1[64:128], rpad),
                  Wf1[128:256], Wf1[256:512])
    nclass = Wc.shape[1]

    pooled_parts = []
    for h in range(H):
        f = x0[h * NH:(h + 1) * NH]
        fs = []
        for (dt, op), (wt, wbot, bias) in zip(dims, prepped):
            pt, idx_t = _tc_layer_a(f, wt, bias, 128, op)
            e = _sc_gather(f, idx_t, 128)
            f = _tc_layer_b(e, f, pt, wbot, 128, op)
            fs.append(f)
        pooled_parts.append(_fc1pool(fs, wf1_splits, bf1[None, :]))

    pooled = jnp.concatenate(pooled_parts, axis=0)
    return _classifier(pooled, Wa, ba[None, :], Wb, bb[None, :],
                       Wc, bc[None, :], nclass)
